# baseline (speedup 1.0000x reference)
import functools

import jax
import jax.numpy as jnp
from jax import lax
from jax.experimental import pallas as pl
from jax.experimental.pallas import tpu as pltpu
from jax.experimental.pallas import tpu_sc as plsc

N = 320000
D = 128
S = 1024
NC = 2
NS = 16
NW = NC * NS
GP = 128
G = N // GP
KB = 2
BR = KB * GP
NBUF = 3
NGT = 80
GPAD = NGT * NW
NBLKT = NGT // KB
SEG_PER_TILE = S // NS
SEG_PER_W = S // NW
BROW = S // GP

_MESH = plsc.VectorSubcoreMesh(
    core_axis_name="c", subcore_axis_name="s", num_cores=NC, num_subcores=NS
)


@functools.partial(
    pl.kernel,
    out_type=[
        jax.ShapeDtypeStruct((NC * S, D), jnp.float32),
        jax.ShapeDtypeStruct((NW * 2, S), jnp.float32),
    ],
    mesh=_MESH,
    compiler_params=pltpu.CompilerParams(needs_layout_passes=False),
    scratch_types=[
        pltpu.VMEM((NBLKT, KB, GP), jnp.int32),
        pltpu.VMEM((NGT * GP,), jnp.int32),
        pltpu.VMEM((NBUF, BR, D), jnp.float32),
        pltpu.VMEM((S,), jnp.float32),
        pltpu.VMEM((S,), jnp.float32),
        pltpu.VMEM((16,), jnp.int32),
        pltpu.VMEM_SHARED((S, D), jnp.float32),
        pltpu.SemaphoreType.DMA((NBUF,)),
        pltpu.SemaphoreType.DMA((NBUF,)),
    ],
)
def _phase1(nf_hbm, bpad_hbm, b3d_hbm, zrow_hbm,
            psum_hbm, pbound_hbm,
            ids_v, idsf_v, rows_v, p_v, e_v, pad_v, acc_sh, sems, ssems):
    cid = lax.axis_index("c")
    sid = lax.axis_index("s")
    gid = sid * NC + cid
    ng = jnp.minimum(NGT, G - gid * NGT)
    nblk = (ng + KB - 1) // KB
    r0 = gid * NGT * GP

    pltpu.sync_copy(b3d_hbm.at[pl.ds(gid * NBLKT, NBLKT)], ids_v)
    pltpu.sync_copy(bpad_hbm.at[pl.ds(r0, NGT * GP)], idsf_v)

    @pl.when(gid == 0)
    def _():
        pad_v[...] = jnp.full((16,), -1, jnp.int32)

    @pl.when(gid > 0)
    def _():
        pltpu.sync_copy(bpad_hbm.at[pl.ds(r0 - 16, 16)], pad_v)

    zero16 = jnp.zeros((16,), jnp.float32)

    def zpe(k, c):
        p_v[pl.ds(k * 16, 16)] = zero16
        e_v[pl.ds(k * 16, 16)] = zero16
        return c

    lax.fori_loop(0, S // 16, zpe, 0)
    zbase = sid * SEG_PER_TILE
    pltpu.sync_copy(zrow_hbm, acc_sh.at[pl.ds(zbase, SEG_PER_TILE)])

    def issue(t, b):
        @pl.when(t < nblk)
        def _():
            pltpu.async_copy(nf_hbm.at[pl.ds(r0 + t * BR, BR)], rows_v.at[b],
                             sems.at[b])

    def wait(t, b):
        pltpu.make_async_copy(nf_hbm.at[pl.ds(r0 + t * BR, BR)],
                              rows_v.at[b], sems.at[b]).wait()

    for b in range(NBUF):
        issue(b, b)

    iota = lax.iota(jnp.int32, 16)
    shift_idx = jnp.maximum(iota - 1, 0)
    b15 = iota * 0 + 15
    lane0 = iota == 0
    prev0 = pad_v[...]
    prev = prev0.at[b15].get(mode="promise_in_bounds")

    def window(w, prev_b):
        v = idsf_v[pl.ds(w * 16, 16)]
        vsh = v.at[shift_idx].get(mode="promise_in_bounds")
        vp = jnp.where(lane0, prev_b, vsh)
        m = v != vp
        posf = (r0 + w * 16 + iota).astype(jnp.float32)
        plsc.store_scatter(p_v, [v], posf, mask=m)
        plsc.store_scatter(e_v, [vp], posf, mask=m & (vp >= 0))
        return v.at[b15].get(mode="promise_in_bounds")

    prev_last = lax.fori_loop(0, ng * (GP // 16), window, prev)

    @pl.when(gid == NW - 1)
    def _():
        endf = jnp.full((16,), float(N), jnp.float32)
        plsc.store_scatter(e_v, [prev_last], endf, mask=lane0)

    plsc.subcore_barrier()

    def scatter(t, b):
        @pl.when(t < nblk)
        def _():
            for j in range(KB):
                pltpu.async_copy(rows_v.at[b, pl.ds(j * GP, GP)],
                                 acc_sh.at[ids_v.at[t, j]], ssems.at[b],
                                 add=True)
            for j in range(KB):
                pltpu.make_async_copy(rows_v.at[b, pl.ds(j * GP, GP)],
                                      acc_sh.at[ids_v.at[t, j]],
                                      ssems.at[b]).wait()

    def sstep(ss, c):
        for b in range(NBUF):
            t = ss * NBUF + b
            @pl.when(t < nblk)
            def _():
                wait(t, b)
            scatter(t, b)
            issue(t + NBUF, b)
        return c

    lax.fori_loop(0, NBLKT // NBUF + 1, sstep, 0)

    plsc.subcore_barrier()

    pltpu.sync_copy(acc_sh.at[pl.ds(zbase, SEG_PER_TILE)],
                    psum_hbm.at[pl.ds(cid * S + zbase, SEG_PER_TILE)])
    pltpu.sync_copy(p_v, pbound_hbm.at[gid * 2])
    pltpu.sync_copy(e_v, pbound_hbm.at[gid * 2 + 1])


@functools.partial(
    pl.kernel,
    out_type=jax.ShapeDtypeStruct((S, D), jnp.float32),
    mesh=_MESH,
    compiler_params=pltpu.CompilerParams(needs_layout_passes=False),
    scratch_types=[
        pltpu.VMEM((SEG_PER_W, D), jnp.float32),
        pltpu.VMEM((SEG_PER_W, D), jnp.float32),
        pltpu.VMEM((NW * 2, S), jnp.float32),
        pltpu.VMEM((SEG_PER_W, D), jnp.float32),
        pltpu.VMEM((SEG_PER_W, D), jnp.float32),
    ],
)
def _phase2(psum_hbm, pbound_hbm, out_hbm, a_v, b_v, pb_v, cnt_v, o_v):
    cid = lax.axis_index("c")
    sid = lax.axis_index("s")
    gid = sid * NC + cid
    seg0 = gid * SEG_PER_W

    pltpu.sync_copy(psum_hbm.at[pl.ds(seg0, SEG_PER_W)], a_v)
    pltpu.sync_copy(psum_hbm.at[pl.ds(S + seg0, SEG_PER_W)], b_v)
    pltpu.sync_copy(pbound_hbm, pb_v)

    zero16 = jnp.zeros((16,), jnp.float32)

    def accw(w, carry):
        p0, p1, e0, e1 = carry
        p0 = p0 + pb_v[w * 2, pl.ds(seg0, 16)]
        p1 = p1 + pb_v[w * 2, pl.ds(seg0 + 16, 16)]
        e0 = e0 + pb_v[w * 2 + 1, pl.ds(seg0, 16)]
        e1 = e1 + pb_v[w * 2 + 1, pl.ds(seg0 + 16, 16)]
        return (p0, p1, e0, e1)

    p0, p1, e0, e1 = lax.fori_loop(
        0, NW, accw, (zero16, zero16, zero16, zero16))
    cnt0 = jnp.maximum(e0 - p0, 1.0)
    cnt1 = jnp.maximum(e1 - p1, 1.0)
    for l in range(16):
        idx = jnp.full((16,), l, jnp.int32)
        cnt_v[l, pl.ds(0, 16)] = cnt0.at[idx].get(mode="promise_in_bounds")
        cnt_v[16 + l, pl.ds(0, 16)] = cnt1.at[idx].get(
            mode="promise_in_bounds")

    def row(i, carry):
        cnt = cnt_v[i, pl.ds(0, 16)]
        for j in range(D // 16):
            sl = pl.ds(j * 16, 16)
            o_v[i, sl] = (a_v[i, sl] + b_v[i, sl]) / cnt
        return carry

    lax.fori_loop(0, SEG_PER_W, row, 0)
    pltpu.sync_copy(o_v, out_hbm.at[pl.ds(seg0, SEG_PER_W)])


def kernel(node_features, batch):
    bpad = jnp.concatenate(
        [batch, jnp.full((GPAD * GP - N,), -1, jnp.int32)])
    b3d = jnp.concatenate(
        [batch, jnp.zeros((GPAD * GP - N,), jnp.int32)]
    ).reshape(NW * NBLKT, KB, GP)
    zrow = jnp.zeros((SEG_PER_TILE, D), jnp.float32)
    psum, pbound = _phase1(node_features, bpad, b3d, zrow)
    return _phase2(psum, pbound)


# speedup vs baseline: 1.0890x; 1.0890x over previous
import functools

import jax
import jax.numpy as jnp
from jax import lax
from jax.experimental import pallas as pl
from jax.experimental.pallas import tpu as pltpu
from jax.experimental.pallas import tpu_sc as plsc

N = 320000
D = 128
S = 1024
NC = 2
NS = 16
NW = NC * NS
GP = 128
G = N // GP
KB = 2
BR = KB * GP
NBUF = 3
NGT = 80
GPAD = NGT * NW
NBLKT = NGT // KB
SEG_PER_TILE = S // NS
SEG_PER_W = S // NW
BROW = S // GP

_MESH = plsc.VectorSubcoreMesh(
    core_axis_name="c", subcore_axis_name="s", num_cores=NC, num_subcores=NS
)


@functools.partial(
    pl.kernel,
    out_type=[
        jax.ShapeDtypeStruct((NC * S, D), jnp.float32),
        jax.ShapeDtypeStruct((NW * 2, S), jnp.float32),
    ],
    mesh=_MESH,
    compiler_params=pltpu.CompilerParams(needs_layout_passes=False),
    scratch_types=[
        pltpu.VMEM((NBLKT, KB, GP), jnp.int32),
        pltpu.VMEM((NGT * GP,), jnp.int32),
        pltpu.VMEM((NBUF, BR, D), jnp.float32),
        pltpu.VMEM((S,), jnp.float32),
        pltpu.VMEM((S,), jnp.float32),
        pltpu.VMEM((16,), jnp.int32),
        pltpu.VMEM_SHARED((S, D), jnp.float32),
        pltpu.SemaphoreType.DMA((NBUF,)),
        pltpu.SemaphoreType.DMA((NBUF,)),
    ],
)
def _phase1(nf_hbm, bpad_hbm, b3d_hbm, zrow_hbm,
            psum_hbm, pbound_hbm,
            ids_v, idsf_v, rows_v, p_v, e_v, pad_v, acc_sh, sems, ssems):
    cid = lax.axis_index("c")
    sid = lax.axis_index("s")
    gid = sid * NC + cid
    ng = jnp.minimum(NGT, G - gid * NGT)
    nblk = (ng + KB - 1) // KB
    r0 = gid * NGT * GP

    pltpu.sync_copy(b3d_hbm.at[pl.ds(gid * NBLKT, NBLKT)], ids_v)
    pltpu.sync_copy(bpad_hbm.at[pl.ds(r0, NGT * GP)], idsf_v)

    @pl.when(gid == 0)
    def _():
        pad_v[...] = jnp.full((16,), -1, jnp.int32)

    @pl.when(gid > 0)
    def _():
        pltpu.sync_copy(bpad_hbm.at[pl.ds(r0 - 16, 16)], pad_v)

    zero16 = jnp.zeros((16,), jnp.float32)

    def zpe(k, c):
        p_v[pl.ds(k * 16, 16)] = zero16
        e_v[pl.ds(k * 16, 16)] = zero16
        return c

    lax.fori_loop(0, S // 16, zpe, 0)
    zbase = sid * SEG_PER_TILE
    pltpu.sync_copy(zrow_hbm, acc_sh.at[pl.ds(zbase, SEG_PER_TILE)])

    def issue(t, b):
        @pl.when(t < nblk)
        def _():
            pltpu.async_copy(nf_hbm.at[pl.ds(r0 + t * BR, BR)], rows_v.at[b],
                             sems.at[b])

    def wait(t, b):
        pltpu.make_async_copy(nf_hbm.at[pl.ds(r0 + t * BR, BR)],
                              rows_v.at[b], sems.at[b]).wait()

    for b in range(NBUF):
        issue(b, b)

    iota = lax.iota(jnp.int32, 16)
    shift_idx = jnp.maximum(iota - 1, 0)
    b15 = iota * 0 + 15
    lane0 = iota == 0
    prev0 = pad_v[...]
    prev = prev0.at[b15].get(mode="promise_in_bounds")

    def window(w, prev_b):
        v = idsf_v[pl.ds(w * 16, 16)]
        vsh = v.at[shift_idx].get(mode="promise_in_bounds")
        vp = jnp.where(lane0, prev_b, vsh)
        m = v != vp
        posf = (r0 + w * 16 + iota).astype(jnp.float32)
        plsc.store_scatter(p_v, [v], posf, mask=m)
        plsc.store_scatter(e_v, [vp], posf, mask=m & (vp >= 0))
        return v.at[b15].get(mode="promise_in_bounds")

    prev_last = lax.fori_loop(0, ng * (GP // 16), window, prev)

    @pl.when(gid == NW - 1)
    def _():
        endf = jnp.full((16,), float(N), jnp.float32)
        plsc.store_scatter(e_v, [prev_last], endf, mask=lane0)

    plsc.subcore_barrier()

    def scatter(t, b):
        @pl.when(t < nblk)
        def _():
            for j in range(KB):
                pltpu.sync_copy(rows_v.at[b, pl.ds(j * GP, GP)],
                                acc_sh.at[ids_v.at[t, j]], add=True)

    def sstep(ss, c):
        for b in range(NBUF):
            t = ss * NBUF + b
            @pl.when(t < nblk)
            def _():
                wait(t, b)
            scatter(t, b)
            issue(t + NBUF, b)
        return c

    lax.fori_loop(0, NBLKT // NBUF + 1, sstep, 0)

    plsc.subcore_barrier()

    pltpu.sync_copy(acc_sh.at[pl.ds(zbase, SEG_PER_TILE)],
                    psum_hbm.at[pl.ds(cid * S + zbase, SEG_PER_TILE)])
    pltpu.sync_copy(p_v, pbound_hbm.at[gid])
    pltpu.sync_copy(e_v, pbound_hbm.at[NW + gid])


def _phase2_body(psum_ref, pbound_ref, out_ref):
    ps = psum_ref[0:S, :] + psum_ref[S:2 * S, :]
    cnt = (jnp.sum(pbound_ref[NW:, :], axis=0)
           - jnp.sum(pbound_ref[:NW, :], axis=0))
    cnt = jnp.maximum(cnt, 1.0)
    out_ref[...] = ps / cnt[:, None]


_phase2 = pl.pallas_call(
    _phase2_body, out_shape=jax.ShapeDtypeStruct((S, D), jnp.float32))


def kernel(node_features, batch):
    bpad = jnp.concatenate(
        [batch, jnp.full((GPAD * GP - N,), -1, jnp.int32)])
    b3d = jnp.concatenate(
        [batch, jnp.zeros((GPAD * GP - N,), jnp.int32)]
    ).reshape(NW * NBLKT, KB, GP)
    zrow = jnp.zeros((SEG_PER_TILE, D), jnp.float32)
    psum, pbound = _phase1(node_features, bpad, b3d, zrow)
    return _phase2(psum, pbound)
